# baseline (speedup 1.0000x reference)
import jax
import jax.numpy as jnp
import numpy as np
from jax.experimental import pallas as pl
from jax.experimental.pallas import tpu as pltpu

_N = 10000
_E = 160000
_DNE = 128
_DED = 16
_H = 8
_CD = 384
_BN = 1000
_BE = 2000


def _dot(a, b):
    return jax.lax.dot_general(a, b, (((1,), (0,)), ((), ())),
                               preferred_element_type=jnp.float32)


def _proj_nodes_kernel(h_ref, wq, bq, wk, bk, wv1, bv1, wv2, bv2,
                       ws1, bs1, ws2, bs2, q_o, k_o, v_o, s_o):
    x = h_ref[...]
    q_o[...] = _dot(x, wq[...]) + bq[...]
    k_o[...] = _dot(x, wk[...]) + bk[...]
    v1 = jax.nn.silu(_dot(x, wv1[...]) + bv1[...])
    v_o[...] = _dot(v1, wv2[...]) + bv2[...]
    s1 = jax.nn.silu(_dot(x, ws1[...]) + bs1[...])
    s_o[...] = _dot(s1, ws2[...]) + bs2[...]


def _pass1_kernel(edge_ref, t_ref, wre, bre, q_ref, k_ref, hsel,
                  alpha_o, m_o, deg_o, geom_s):
    @pl.when(pl.program_id(0) == 0)
    def _():
        m_o[...] = jnp.full_like(m_o, -1e9)
        deg_o[...] = jnp.zeros_like(deg_o)

    geom_s[...] = jax.nn.silu(_dot(t_ref[...], wre[...]) + bre[...])

    def body(e, carry):
        p = edge_ref[0, 0, e]
        i = p // 16384
        j = p - i * 16384
        prod = q_ref[pl.ds(i, 1), :] * k_ref[pl.ds(j, 1), :] * geom_s[pl.ds(e, 1), :]
        a = _dot(prod, hsel[...])
        alpha_o[pl.ds(e, 1), :] = a
        m_o[pl.ds(i, 1), :] = jnp.maximum(m_o[pl.ds(i, 1), :], a)
        deg_o[pl.ds(i, 1), :] = deg_o[pl.ds(i, 1), :] + 1.0
        return carry

    jax.lax.fori_loop(0, _BE, body, 0)


def _pass2_kernel(edge_ref, alpha_ref, m_ref, ex_o, den_o):
    @pl.when(pl.program_id(0) == 0)
    def _():
        den_o[...] = jnp.zeros_like(den_o)

    def body(e, carry):
        p = edge_ref[0, 0, e]
        i = p // 16384
        exr = jnp.exp(alpha_ref[pl.ds(e, 1), :] - m_ref[pl.ds(i, 1), :])
        ex_o[pl.ds(e, 1), :] = exr
        den_o[pl.ds(i, 1), :] = den_o[pl.ds(i, 1), :] + exr
        return carry

    jax.lax.fori_loop(0, _BE, body, 0)


def _attn_kernel(edge_ref, ex_ref, den_ref, deg_ref, attn_o):
    inv_sqrt_d = np.float32(1.0 / np.sqrt(float(_DNE)))

    def body(e, carry):
        p = edge_ref[0, 0, e]
        i = p // 16384
        den = jnp.maximum(den_ref[pl.ds(i, 1), :], 1e-12)
        deg = deg_ref[pl.ds(i, 1), :]
        attn_o[pl.ds(e, 1), :] = (ex_ref[pl.ds(e, 1), :] / den
                                  * jnp.sqrt(deg) * inv_sqrt_d)
        return carry

    jax.lax.fori_loop(0, _BE, body, 0)


def _coeff_rows(edge_ref, t_ref, wrs, brs, c_ref, attn_ref, v_ref, s_ref,
                asel, tpc_s, e):
    p = edge_ref[0, 0, e]
    i = p // 16384
    j = p - i * 16384
    a128 = _dot(attn_ref[pl.ds(e, 1), :], asel[...])
    cc = (a128 * v_ref[pl.ds(j, 1), :]
          + tpc_s[pl.ds(e, 1), :] * s_ref[pl.ds(j, 1), :])
    return i, j, cc


def _scat_h_kernel(edge_ref, t_ref, wrs, brs, c_ref, attn_ref, v_ref, s_ref,
                   h_ref, asel, out_o, tpc_s):
    @pl.when(pl.program_id(0) == 0)
    def _():
        out_o[...] = h_ref[...]

    tpc_s[...] = (_dot(t_ref[...], wrs[...]) + brs[...]) * c_ref[...]

    def body(e, carry):
        i, j, cc = _coeff_rows(edge_ref, t_ref, wrs, brs, c_ref, attn_ref,
                               v_ref, s_ref, asel, tpc_s, e)
        out_o[pl.ds(i, 1), :] = out_o[pl.ds(i, 1), :] + cc
        return carry

    jax.lax.fori_loop(0, _BE, body, 0)


def _scat_dir_kernel(edge_ref, t_ref, wrs, brs, c_ref, attn_ref, v_ref, s_ref,
                     rl_ref, asel, expander, out_o, tpc_s, rl3_s):
    @pl.when(pl.program_id(0) == 0)
    def _():
        out_o[...] = jnp.zeros_like(out_o)

    tpc_s[...] = (_dot(t_ref[...], wrs[...]) + brs[...]) * c_ref[...]
    rl3_s[...] = _dot(rl_ref[...], expander[...])

    def body(e, carry):
        i, j, cc = _coeff_rows(edge_ref, t_ref, wrs, brs, c_ref, attn_ref,
                               v_ref, s_ref, asel, tpc_s, e)
        cc3 = jnp.concatenate([cc, cc, cc], axis=1)
        out_o[pl.ds(i, 1), :] = (out_o[pl.ds(i, 1), :]
                                 + rl3_s[pl.ds(e, 1), :] * cc3)
        return carry

    jax.lax.fori_loop(0, _BE, body, 0)


def _scat_tens_kernel(edge_ref, t_ref, wrs, brs, c_ref, attn_ref, v_ref,
                      s_ref, x_ref, asel, out_o, tpc_s):
    @pl.when(pl.program_id(0) == 0)
    def _():
        out_o[...] = jnp.zeros_like(out_o)

    tpc_s[...] = (_dot(t_ref[...], wrs[...]) + brs[...]) * c_ref[...]

    def body(e, carry):
        i, j, cc = _coeff_rows(edge_ref, t_ref, wrs, brs, c_ref, attn_ref,
                               v_ref, s_ref, asel, tpc_s, e)
        cc3 = jnp.concatenate([cc, cc, cc], axis=1)
        out_o[pl.ds(i, 1), :] = (out_o[pl.ds(i, 1), :]
                                 + x_ref[pl.ds(j, 1), :] * cc3)
        return carry

    jax.lax.fori_loop(0, _BE, body, 0)


def _add3_kernel(a_ref, b_ref, d_ref, o_ref):
    o_ref[...] = a_ref[...] + b_ref[...] + d_ref[...]


def kernel(h, t, X_list, edge, rtilde, c, last_layer, W_q, b_q, W_k, b_k,
           W_re, b_re, Wv1, bv1, Wv2, bv2, W_rs, b_rs, Ws1, bs1, Ws2, bs2):
    f32 = jnp.float32
    nb = _N // _BN
    eb = _E // _BE

    wq_t, wk_t = W_q.T, W_k.T
    wv1_t, wv2_t = Wv1.T, Wv2.T
    ws1_t, ws2_t = Ws1.T, Ws2.T
    wre_t, wrs_t = W_re.T, W_rs.T
    bq = b_q.reshape(1, _DNE)
    bk = b_k.reshape(1, _DNE)
    bv1 = bv1.reshape(1, _DNE)
    bv2 = bv2.reshape(1, _CD)
    bs1 = bs1.reshape(1, _DNE)
    bs2 = bs2.reshape(1, _CD)
    bre = b_re.reshape(1, _DNE)
    brs = b_rs.reshape(1, _CD)
    x_flat = X_list[0].reshape(_N, _CD)
    rl = rtilde[1]
    ep = (edge[:, 0] * 16384 + edge[:, 1]).reshape(eb, 1, _BE)

    hsel = (jax.lax.broadcasted_iota(jnp.int32, (_DNE, _H), 0) // 16
            == jax.lax.broadcasted_iota(jnp.int32, (_DNE, _H), 1)).astype(f32)
    asel = (jax.lax.broadcasted_iota(jnp.int32, (_H, _CD), 1) // 48
            == jax.lax.broadcasted_iota(jnp.int32, (_H, _CD), 0)).astype(f32)
    expander = (jax.lax.broadcasted_iota(jnp.int32, (3, _CD), 1) // 128
                == jax.lax.broadcasted_iota(jnp.int32, (3, _CD), 0)).astype(f32)

    full = lambda shape: pl.BlockSpec(shape, lambda b: (0, 0))
    blk = lambda shape: pl.BlockSpec(shape, lambda b: (b, 0))
    edge_spec = pl.BlockSpec((1, 1, _BE), lambda b: (b, 0, 0),
                             memory_space=pltpu.SMEM)

    q_n, k_n, v_n, s_n = pl.pallas_call(
        _proj_nodes_kernel,
        grid=(nb,),
        in_specs=[blk((_BN, _DNE))] + [full(w.shape) for w in
                  (wq_t, bq, wk_t, bk, wv1_t, bv1, wv2_t, bv2,
                   ws1_t, bs1, ws2_t, bs2)],
        out_specs=[blk((_BN, _DNE)), blk((_BN, _DNE)),
                   blk((_BN, _CD)), blk((_BN, _CD))],
        out_shape=[jax.ShapeDtypeStruct((_N, _DNE), f32),
                   jax.ShapeDtypeStruct((_N, _DNE), f32),
                   jax.ShapeDtypeStruct((_N, _CD), f32),
                   jax.ShapeDtypeStruct((_N, _CD), f32)],
    )(h, wq_t, bq, wk_t, bk, wv1_t, bv1, wv2_t, bv2, ws1_t, bs1, ws2_t, bs2)

    alpha, m_tab, deg_tab = pl.pallas_call(
        _pass1_kernel,
        grid=(eb,),
        in_specs=[edge_spec, blk((_BE, _DED)), full((_DED, _DNE)),
                  full((1, _DNE)), full((_N, _DNE)), full((_N, _DNE)),
                  full((_DNE, _H))],
        out_specs=[blk((_BE, _H)), full((_N, _H)), full((_N, _H))],
        out_shape=[jax.ShapeDtypeStruct((_E, _H), f32),
                   jax.ShapeDtypeStruct((_N, _H), f32),
                   jax.ShapeDtypeStruct((_N, _H), f32)],
        scratch_shapes=[pltpu.VMEM((_BE, _DNE), f32)],
    )(ep, t, wre_t, bre, q_n, k_n, hsel)

    ex, den_tab = pl.pallas_call(
        _pass2_kernel,
        grid=(eb,),
        in_specs=[edge_spec, blk((_BE, _H)), full((_N, _H))],
        out_specs=[blk((_BE, _H)), full((_N, _H))],
        out_shape=[jax.ShapeDtypeStruct((_E, _H), f32),
                   jax.ShapeDtypeStruct((_N, _H), f32)],
    )(ep, alpha, m_tab)

    attn = pl.pallas_call(
        _attn_kernel,
        grid=(eb,),
        in_specs=[edge_spec, blk((_BE, _H)), full((_N, _H)), full((_N, _H))],
        out_specs=blk((_BE, _H)),
        out_shape=jax.ShapeDtypeStruct((_E, _H), f32),
    )(ep, ex, den_tab, deg_tab)

    comp = lambda a, k: a[:, 128 * k:128 * (k + 1)]
    wcomp = lambda k: wrs_t[:, 128 * k:128 * (k + 1)]
    bcomp = lambda k: brs[:, 128 * k:128 * (k + 1)]
    aselc = [
        (jax.lax.broadcasted_iota(jnp.int32, (_H, _DNE), 0)
         == (jax.lax.broadcasted_iota(jnp.int32, (_H, _DNE), 1) + 128 * k) // 48
         ).astype(f32)
        for k in range(3)
    ]

    scat_in = lambda extra: ([edge_spec, blk((_BE, _DED)), full((_DED, _DNE)),
                              full((1, _DNE)), blk((_BE, 1)), blk((_BE, _H)),
                              full((_N, _DNE)), full((_N, _DNE))] + extra)

    h_out = pl.pallas_call(
        _scat_h_kernel,
        grid=(eb,),
        in_specs=scat_in([full((_N, _DNE)), full((_H, _DNE))]),
        out_specs=full((_N, _DNE)),
        out_shape=jax.ShapeDtypeStruct((_N, _DNE), f32),
        scratch_shapes=[pltpu.VMEM((_BE, _DNE), f32)],
    )(ep, t, wcomp(0), bcomp(0), c, attn, comp(v_n, 0), comp(s_n, 0),
      h, aselc[0])

    x_dir = pl.pallas_call(
        _scat_dir_kernel,
        grid=(eb,),
        in_specs=scat_in([blk((_BE, 3)), full((_H, _DNE)), full((3, _CD))]),
        out_specs=full((_N, _CD)),
        out_shape=jax.ShapeDtypeStruct((_N, _CD), f32),
        scratch_shapes=[pltpu.VMEM((_BE, _DNE), f32),
                        pltpu.VMEM((_BE, _CD), f32)],
    )(ep, t, wcomp(1), bcomp(1), c, attn, comp(v_n, 1), comp(s_n, 1),
      rl, aselc[1], expander)

    x_tens = pl.pallas_call(
        _scat_tens_kernel,
        grid=(eb,),
        in_specs=scat_in([full((_N, _CD)), full((_H, _DNE))]),
        out_specs=full((_N, _CD)),
        out_shape=jax.ShapeDtypeStruct((_N, _CD), f32),
        scratch_shapes=[pltpu.VMEM((_BE, _DNE), f32)],
    )(ep, t, wcomp(2), bcomp(2), c, attn, comp(v_n, 2), comp(s_n, 2),
      x_flat, aselc[2])

    x_out = pl.pallas_call(
        _add3_kernel,
        grid=(nb,),
        in_specs=[blk((_BN, _CD))] * 3,
        out_specs=blk((_BN, _CD)),
        out_shape=jax.ShapeDtypeStruct((_N, _CD), f32),
    )(x_flat, x_dir, x_tens)

    return (h_out, t, x_out.reshape(_N, 3, _DNE))


# speedup vs baseline: 9.3249x; 9.3249x over previous
import jax
import jax.numpy as jnp
import numpy as np
from jax.experimental import pallas as pl
from jax.experimental.pallas import tpu as pltpu

_N = 10000
_E = 160000
_DNE = 128
_DED = 16
_H = 8
_CD = 384
_BN = 1000
_BE = 2000


def _dot(a, b):
    return jax.lax.dot_general(a, b, (((1,), (0,)), ((), ())),
                               preferred_element_type=jnp.float32)


def _proj_nodes_kernel(h_ref, wq, bq, wk, bk, wv1, bv1, wv2, bv2,
                       ws1, bs1, ws2, bs2, q_o, k_o, v_o, s_o):
    x = h_ref[...]
    q_o[...] = _dot(x, wq[...]) + bq[...]
    k_o[...] = _dot(x, wk[...]) + bk[...]
    v1 = jax.nn.silu(_dot(x, wv1[...]) + bv1[...])
    v_o[...] = _dot(v1, wv2[...]) + bv2[...]
    s1 = jax.nn.silu(_dot(x, ws1[...]) + bs1[...])
    s_o[...] = _dot(s1, ws2[...]) + bs2[...]


def _pass1_kernel(edge_ref, t_ref, wre, bre, q_ref, k_ref, hsel,
                  alpha_o, m_o, deg_o, geom_s, qg_s, kg_s):
    @pl.when(pl.program_id(0) == 0)
    def _():
        m_o[...] = jnp.full_like(m_o, -1e9)
        deg_o[...] = jnp.zeros_like(deg_o)

    geom_s[...] = jax.nn.silu(_dot(t_ref[...], wre[...]) + bre[...])

    def gather(e, carry):
        p = edge_ref[0, 0, e]
        i = p // 16384
        j = p - i * 16384
        qg_s[pl.ds(e, 1), :] = q_ref[pl.ds(i, 1), :]
        kg_s[pl.ds(e, 1), :] = k_ref[pl.ds(j, 1), :]
        return carry

    jax.lax.fori_loop(0, _BE, gather, 0, unroll=8)

    alpha_o[...] = _dot(qg_s[...] * kg_s[...] * geom_s[...], hsel[...])

    def rmw(e, carry):
        p = edge_ref[0, 0, e]
        i = p // 16384
        a = alpha_o[pl.ds(e, 1), :]
        m_o[pl.ds(i, 1), :] = jnp.maximum(m_o[pl.ds(i, 1), :], a)
        deg_o[pl.ds(i, 1), :] = deg_o[pl.ds(i, 1), :] + 1.0
        return carry

    jax.lax.fori_loop(0, _BE, rmw, 0, unroll=8)


def _pass2_kernel(edge_ref, alpha_ref, m_ref, ex_o, den_o):
    @pl.when(pl.program_id(0) == 0)
    def _():
        den_o[...] = jnp.zeros_like(den_o)

    def body(e, carry):
        p = edge_ref[0, 0, e]
        i = p // 16384
        exr = jnp.exp(alpha_ref[pl.ds(e, 1), :] - m_ref[pl.ds(i, 1), :])
        ex_o[pl.ds(e, 1), :] = exr
        den_o[pl.ds(i, 1), :] = den_o[pl.ds(i, 1), :] + exr
        return carry

    jax.lax.fori_loop(0, _BE, body, 0, unroll=8)


def _attn_kernel(edge_ref, ex_ref, den_ref, deg_ref, attn_o):
    inv_sqrt_d = np.float32(1.0 / np.sqrt(float(_DNE)))

    def body(e, carry):
        p = edge_ref[0, 0, e]
        i = p // 16384
        den = jnp.maximum(den_ref[pl.ds(i, 1), :], 1e-12)
        deg = deg_ref[pl.ds(i, 1), :]
        attn_o[pl.ds(e, 1), :] = (ex_ref[pl.ds(e, 1), :] / den
                                  * jnp.sqrt(deg) * inv_sqrt_d)
        return carry

    jax.lax.fori_loop(0, _BE, body, 0, unroll=8)


def _coeff_rows(edge_ref, a128_s, tpc_s, v_ref, s_ref, e):
    p = edge_ref[0, 0, e]
    i = p // 16384
    j = p - i * 16384
    cc = (a128_s[pl.ds(e, 1), :] * v_ref[pl.ds(j, 1), :]
          + tpc_s[pl.ds(e, 1), :] * s_ref[pl.ds(j, 1), :])
    return i, j, cc


def _scat_h_kernel(edge_ref, t_ref, wrs, brs, c_ref, attn_ref, v_ref, s_ref,
                   h_ref, asel, out_o, tpc_s, a128_s):
    @pl.when(pl.program_id(0) == 0)
    def _():
        out_o[...] = h_ref[...]

    tpc_s[...] = (_dot(t_ref[...], wrs[...]) + brs[...]) * c_ref[...]
    a128_s[...] = _dot(attn_ref[...], asel[...])

    def body(e, carry):
        i, j, cc = _coeff_rows(edge_ref, a128_s, tpc_s, v_ref, s_ref, e)
        out_o[pl.ds(i, 1), :] = out_o[pl.ds(i, 1), :] + cc
        return carry

    jax.lax.fori_loop(0, _BE, body, 0, unroll=8)


def _scat_dir_kernel(edge_ref, t_ref, wrs, brs, c_ref, attn_ref, v_ref, s_ref,
                     rl_ref, asel, expander, out_o, tpc_s, rl3_s, a128_s):
    @pl.when(pl.program_id(0) == 0)
    def _():
        out_o[...] = jnp.zeros_like(out_o)

    tpc_s[...] = (_dot(t_ref[...], wrs[...]) + brs[...]) * c_ref[...]
    rl3_s[...] = _dot(rl_ref[...], expander[...])
    a128_s[...] = _dot(attn_ref[...], asel[...])

    def body(e, carry):
        i, j, cc = _coeff_rows(edge_ref, a128_s, tpc_s, v_ref, s_ref, e)
        cc3 = jnp.concatenate([cc, cc, cc], axis=1)
        out_o[pl.ds(i, 1), :] = (out_o[pl.ds(i, 1), :]
                                 + rl3_s[pl.ds(e, 1), :] * cc3)
        return carry

    jax.lax.fori_loop(0, _BE, body, 0, unroll=8)


def _scat_tens_kernel(edge_ref, t_ref, wrs, brs, c_ref, attn_ref, v_ref,
                      s_ref, x_ref, asel, out_o, tpc_s, a128_s):
    @pl.when(pl.program_id(0) == 0)
    def _():
        out_o[...] = jnp.zeros_like(out_o)

    tpc_s[...] = (_dot(t_ref[...], wrs[...]) + brs[...]) * c_ref[...]
    a128_s[...] = _dot(attn_ref[...], asel[...])

    def body(e, carry):
        i, j, cc = _coeff_rows(edge_ref, a128_s, tpc_s, v_ref, s_ref, e)
        cc3 = jnp.concatenate([cc, cc, cc], axis=1)
        out_o[pl.ds(i, 1), :] = (out_o[pl.ds(i, 1), :]
                                 + x_ref[pl.ds(j, 1), :] * cc3)
        return carry

    jax.lax.fori_loop(0, _BE, body, 0, unroll=8)


def _add3_kernel(a_ref, b_ref, d_ref, o_ref):
    o_ref[...] = a_ref[...] + b_ref[...] + d_ref[...]


def kernel(h, t, X_list, edge, rtilde, c, last_layer, W_q, b_q, W_k, b_k,
           W_re, b_re, Wv1, bv1, Wv2, bv2, W_rs, b_rs, Ws1, bs1, Ws2, bs2):
    f32 = jnp.float32
    nb = _N // _BN
    eb = _E // _BE

    wq_t, wk_t = W_q.T, W_k.T
    wv1_t, wv2_t = Wv1.T, Wv2.T
    ws1_t, ws2_t = Ws1.T, Ws2.T
    wre_t, wrs_t = W_re.T, W_rs.T
    bq = b_q.reshape(1, _DNE)
    bk = b_k.reshape(1, _DNE)
    bv1 = bv1.reshape(1, _DNE)
    bv2 = bv2.reshape(1, _CD)
    bs1 = bs1.reshape(1, _DNE)
    bs2 = bs2.reshape(1, _CD)
    bre = b_re.reshape(1, _DNE)
    brs = b_rs.reshape(1, _CD)
    x_flat = X_list[0].reshape(_N, _CD)
    rl = rtilde[1]
    ep = (edge[:, 0] * 16384 + edge[:, 1]).reshape(eb, 1, _BE)

    hsel = (jax.lax.broadcasted_iota(jnp.int32, (_DNE, _H), 0) // 16
            == jax.lax.broadcasted_iota(jnp.int32, (_DNE, _H), 1)).astype(f32)
    asel = (jax.lax.broadcasted_iota(jnp.int32, (_H, _CD), 1) // 48
            == jax.lax.broadcasted_iota(jnp.int32, (_H, _CD), 0)).astype(f32)
    expander = (jax.lax.broadcasted_iota(jnp.int32, (3, _CD), 1) // 128
                == jax.lax.broadcasted_iota(jnp.int32, (3, _CD), 0)).astype(f32)

    full = lambda shape: pl.BlockSpec(shape, lambda b: (0, 0))
    blk = lambda shape: pl.BlockSpec(shape, lambda b: (b, 0))
    edge_spec = pl.BlockSpec((1, 1, _BE), lambda b: (b, 0, 0),
                             memory_space=pltpu.SMEM)

    q_n, k_n, v_n, s_n = pl.pallas_call(
        _proj_nodes_kernel,
        grid=(nb,),
        in_specs=[blk((_BN, _DNE))] + [full(w.shape) for w in
                  (wq_t, bq, wk_t, bk, wv1_t, bv1, wv2_t, bv2,
                   ws1_t, bs1, ws2_t, bs2)],
        out_specs=[blk((_BN, _DNE)), blk((_BN, _DNE)),
                   blk((_BN, _CD)), blk((_BN, _CD))],
        out_shape=[jax.ShapeDtypeStruct((_N, _DNE), f32),
                   jax.ShapeDtypeStruct((_N, _DNE), f32),
                   jax.ShapeDtypeStruct((_N, _CD), f32),
                   jax.ShapeDtypeStruct((_N, _CD), f32)],
    )(h, wq_t, bq, wk_t, bk, wv1_t, bv1, wv2_t, bv2, ws1_t, bs1, ws2_t, bs2)

    alpha, m_tab, deg_tab = pl.pallas_call(
        _pass1_kernel,
        grid=(eb,),
        in_specs=[edge_spec, blk((_BE, _DED)), full((_DED, _DNE)),
                  full((1, _DNE)), full((_N, _DNE)), full((_N, _DNE)),
                  full((_DNE, _H))],
        out_specs=[blk((_BE, _H)), full((_N, _H)), full((_N, _H))],
        out_shape=[jax.ShapeDtypeStruct((_E, _H), f32),
                   jax.ShapeDtypeStruct((_N, _H), f32),
                   jax.ShapeDtypeStruct((_N, _H), f32)],
        scratch_shapes=[pltpu.VMEM((_BE, _DNE), f32),
                        pltpu.VMEM((_BE, _DNE), f32),
                        pltpu.VMEM((_BE, _DNE), f32)],
    )(ep, t, wre_t, bre, q_n, k_n, hsel)

    ex, den_tab = pl.pallas_call(
        _pass2_kernel,
        grid=(eb,),
        in_specs=[edge_spec, blk((_BE, _H)), full((_N, _H))],
        out_specs=[blk((_BE, _H)), full((_N, _H))],
        out_shape=[jax.ShapeDtypeStruct((_E, _H), f32),
                   jax.ShapeDtypeStruct((_N, _H), f32)],
    )(ep, alpha, m_tab)

    attn = pl.pallas_call(
        _attn_kernel,
        grid=(eb,),
        in_specs=[edge_spec, blk((_BE, _H)), full((_N, _H)), full((_N, _H))],
        out_specs=blk((_BE, _H)),
        out_shape=jax.ShapeDtypeStruct((_E, _H), f32),
    )(ep, ex, den_tab, deg_tab)

    comp = lambda a, k: a[:, 128 * k:128 * (k + 1)]
    wcomp = lambda k: wrs_t[:, 128 * k:128 * (k + 1)]
    bcomp = lambda k: brs[:, 128 * k:128 * (k + 1)]
    aselc = [
        (jax.lax.broadcasted_iota(jnp.int32, (_H, _DNE), 0)
         == (jax.lax.broadcasted_iota(jnp.int32, (_H, _DNE), 1) + 128 * k) // 48
         ).astype(f32)
        for k in range(3)
    ]

    scat_in = lambda extra: ([edge_spec, blk((_BE, _DED)), full((_DED, _DNE)),
                              full((1, _DNE)), blk((_BE, 1)), blk((_BE, _H)),
                              full((_N, _DNE)), full((_N, _DNE))] + extra)

    h_out = pl.pallas_call(
        _scat_h_kernel,
        grid=(eb,),
        in_specs=scat_in([full((_N, _DNE)), full((_H, _DNE))]),
        out_specs=full((_N, _DNE)),
        out_shape=jax.ShapeDtypeStruct((_N, _DNE), f32),
        scratch_shapes=[pltpu.VMEM((_BE, _DNE), f32),
                        pltpu.VMEM((_BE, _DNE), f32)],
    )(ep, t, wcomp(0), bcomp(0), c, attn, comp(v_n, 0), comp(s_n, 0),
      h, aselc[0])

    x_dir = pl.pallas_call(
        _scat_dir_kernel,
        grid=(eb,),
        in_specs=scat_in([blk((_BE, 3)), full((_H, _DNE)), full((3, _CD))]),
        out_specs=full((_N, _CD)),
        out_shape=jax.ShapeDtypeStruct((_N, _CD), f32),
        scratch_shapes=[pltpu.VMEM((_BE, _DNE), f32),
                        pltpu.VMEM((_BE, _CD), f32),
                        pltpu.VMEM((_BE, _DNE), f32)],
    )(ep, t, wcomp(1), bcomp(1), c, attn, comp(v_n, 1), comp(s_n, 1),
      rl, aselc[1], expander)

    x_tens = pl.pallas_call(
        _scat_tens_kernel,
        grid=(eb,),
        in_specs=scat_in([full((_N, _CD)), full((_H, _DNE))]),
        out_specs=full((_N, _CD)),
        out_shape=jax.ShapeDtypeStruct((_N, _CD), f32),
        scratch_shapes=[pltpu.VMEM((_BE, _DNE), f32),
                        pltpu.VMEM((_BE, _DNE), f32)],
    )(ep, t, wcomp(2), bcomp(2), c, attn, comp(v_n, 2), comp(s_n, 2),
      x_flat, aselc[2])

    x_out = pl.pallas_call(
        _add3_kernel,
        grid=(nb,),
        in_specs=[blk((_BN, _CD))] * 3,
        out_specs=blk((_BN, _CD)),
        out_shape=jax.ShapeDtypeStruct((_N, _CD), f32),
    )(x_flat, x_dir, x_tens)

    return (h_out, t, x_out.reshape(_N, 3, _DNE))
